# baseline (speedup 1.0000x reference)
import functools

import jax
import jax.numpy as jnp
from jax import lax
from jax.experimental import pallas as pl
from jax.experimental.pallas import tpu as pltpu
from jax.experimental.pallas import tpu_sc as plsc

N_NODES = 10000
N_EDGES = 160000
D = 256
H = 128

NS = 16
CHUNK = 64
NCHT = -(-(N_EDGES // NS) // CHUNK)
NBUF = 5
IBUF = 6
EPT = NCHT * CHUNK
E_PAD = EPT * NS
ROWS_PT = 624
ROWS_LAST = N_NODES - (NS - 1) * ROWS_PT
ACC_ROWS = N_NODES + 16

_sc_mesh = plsc.VectorSubcoreMesh(core_axis_name="c", subcore_axis_name="s")


@functools.partial(
    pl.kernel,
    out_type=[jax.ShapeDtypeStruct((N_NODES, H), jnp.float32),
              jax.ShapeDtypeStruct((N_NODES, H), jnp.float32)],
    mesh=_sc_mesh,
    scratch_types=[
        pltpu.VMEM((IBUF, CHUNK), jnp.int32),
        pltpu.VMEM((IBUF, CHUNK), jnp.int32),
        pltpu.VMEM((NBUF, CHUNK, H), jnp.float32),
        pltpu.VMEM_SHARED((ACC_ROWS, H), jnp.float32),
        pltpu.SemaphoreType.DMA,
        pltpu.SemaphoreType.DMA,
        pltpu.SemaphoreType.DMA,
    ],
)
def _sc_aggregate(t0_hbm, t1_hbm, src_hbm, dst_hbm, out0_hbm, out1_hbm,
                  srcb, dstb, rows_v, acc, gsem, ssem, isem):
    c = lax.axis_index("c")
    s = lax.axis_index("s")
    r0 = s * ROWS_PT

    def idx_desc(j, b):
        return (pltpu.make_async_copy(src_hbm.at[s, j], srcb.at[b], isem),
                pltpu.make_async_copy(dst_hbm.at[s, j], dstb.at[b], isem))

    def start_gather(j_b, b):
        @pl.when(c == 0)
        def _():
            pltpu.make_async_copy(t0_hbm.at[srcb.at[j_b]],
                                  rows_v.at[b], gsem).start()

        @pl.when(c == 1)
        def _():
            pltpu.make_async_copy(t1_hbm.at[srcb.at[j_b]],
                                  rows_v.at[b], gsem).start()

    def wait_gather(j_b, b):
        pltpu.make_async_copy(t0_hbm.at[srcb.at[j_b]],
                              rows_v.at[b], gsem).wait()

    def scatter_desc(j_b, b):
        return pltpu.make_async_copy(rows_v.at[b], acc.at[dstb.at[j_b]],
                                     ssem)

    def wait_idx(j, b):
        for d in idx_desc(j, b):
            d.wait()

    def zrow(r, carry):
        for i in range(H // 16):
            rows_v[0, r, pl.ds(i * 16, 16)] = jnp.zeros((16,), jnp.float32)
        return carry

    lax.fori_loop(0, CHUNK, zrow, 0)

    def emit_zero(nrows):
        descs = []
        for k in range(nrows // CHUNK):
            descs.append(pltpu.make_async_copy(
                rows_v.at[0], acc.at[pl.ds(r0 + k * CHUNK, CHUNK)], ssem))
        tail = nrows % CHUNK
        if tail:
            descs.append(pltpu.make_async_copy(
                rows_v.at[0, pl.ds(0, tail)],
                acc.at[pl.ds(r0 + (nrows // CHUNK) * CHUNK, tail)], ssem))
        for d in descs:
            d.start()
        for d in descs:
            d.wait()

    @pl.when(s < NS - 1)
    def _():
        emit_zero(ROWS_PT)

    @pl.when(s == NS - 1)
    def _():
        emit_zero(ROWS_LAST)

    for k in range(IBUF - 1):
        for d in idx_desc(k, k):
            d.start()
    wait_idx(0, 0)
    wait_idx(1, 1)
    wait_idx(2, 2)
    wait_idx(3, 3)
    plsc.subcore_barrier()
    start_gather(0, 0)
    start_gather(1, 1)
    start_gather(2, 2)
    start_gather(3, 3)

    def body(j, carry):
        b = lax.rem(j, NBUF)
        jb = lax.rem(j, IBUF)
        wait_gather(jb, b)
        pltpu.async_copy(rows_v.at[b], acc.at[dstb.at[jb]], ssem, add=True)

        @pl.when(j >= 1)
        def _():
            scatter_desc(lax.rem(j + IBUF - 1, IBUF),
                         lax.rem(j + NBUF - 1, NBUF)).wait()

        @pl.when(j + IBUF - 1 <= NCHT - 1)
        def _():
            for d in idx_desc(j + IBUF - 1, lax.rem(j + IBUF - 1, IBUF)):
                d.start()

        @pl.when(j + 4 <= NCHT - 1)
        def _():
            nb = lax.rem(j + 4, IBUF)
            wait_idx(j + 4, nb)
            start_gather(nb, lax.rem(j + 4, NBUF))
        return carry

    lax.fori_loop(0, NCHT, body, 0)
    scatter_desc(lax.rem(NCHT - 1, IBUF), lax.rem(NCHT - 1, NBUF)).wait()
    plsc.subcore_barrier()

    def emit_out(out_hbm, nrows):
        pltpu.sync_copy(acc.at[pl.ds(r0, nrows)],
                        out_hbm.at[pl.ds(r0, nrows)])

    @pl.when((c == 0) & (s < NS - 1))
    def _():
        emit_out(out0_hbm, ROWS_PT)

    @pl.when((c == 0) & (s == NS - 1))
    def _():
        emit_out(out0_hbm, ROWS_LAST)

    @pl.when((c == 1) & (s < NS - 1))
    def _():
        emit_out(out1_hbm, ROWS_PT)

    @pl.when((c == 1) & (s == NS - 1))
    def _():
        emit_out(out1_hbm, ROWS_LAST)


def _mlp2_body(x0_ref, x1_ref, a0_ref, a1_ref, wa_ref, ba_ref, wb_ref, bb_ref,
               o0_ref, o1_ref):
    g0 = x0_ref[...] + a0_ref[...]
    g1 = x1_ref[...] + a1_ref[...]
    wa = wa_ref[...]
    t = jnp.dot(g0, wa[:H], preferred_element_type=jnp.float32)
    t += jnp.dot(g1, wa[H:], preferred_element_type=jnp.float32)
    t = jnp.maximum(t + ba_ref[...], 0.0)
    u = jnp.dot(t, wb_ref[...], preferred_element_type=jnp.float32)
    u = jnp.maximum(u + bb_ref[...], 0.0)
    o0_ref[...] = u[:, :H]
    o1_ref[...] = u[:, H:]


def _mlp3_body(x0_ref, x1_ref, a0_ref, a1_ref, wa_ref, ba_ref, wb_ref, bb_ref,
               wl_ref, bl_ref, o_ref):
    g0 = x0_ref[...] + a0_ref[...]
    g1 = x1_ref[...] + a1_ref[...]
    wa = wa_ref[...]
    t = jnp.dot(g0, wa[:H], preferred_element_type=jnp.float32)
    t += jnp.dot(g1, wa[H:], preferred_element_type=jnp.float32)
    t = jnp.maximum(t + ba_ref[...], 0.0)
    u = jnp.dot(t, wb_ref[...], preferred_element_type=jnp.float32)
    u = jnp.maximum(u + bb_ref[...], 0.0)
    o_ref[...] = jnp.dot(u, wl_ref[...], preferred_element_type=jnp.float32) + bl_ref[...]


_ROW_BLK = 1000
_h_spec = pl.BlockSpec((_ROW_BLK, H), lambda i: (i, 0))
_d_spec = pl.BlockSpec((_ROW_BLK, D), lambda i: (i, 0))
_w_spec = pl.BlockSpec((D, D), lambda i: (0, 0))
_b_spec = pl.BlockSpec((1, D), lambda i: (0, 0))


def _mlp2(x0, x1, a0, a1, wa, ba, wb, bb):
    return pl.pallas_call(
        _mlp2_body,
        grid=(N_NODES // _ROW_BLK,),
        in_specs=[_h_spec, _h_spec, _h_spec, _h_spec,
                  _w_spec, _b_spec, _w_spec, _b_spec],
        out_specs=[_h_spec, _h_spec],
        out_shape=[jax.ShapeDtypeStruct((N_NODES, H), jnp.float32),
                   jax.ShapeDtypeStruct((N_NODES, H), jnp.float32)],
    )(x0, x1, a0, a1, wa, ba, wb, bb)


def _mlp3(x0, x1, a0, a1, wa, ba, wb, bb, wl, bl):
    return pl.pallas_call(
        _mlp3_body,
        grid=(N_NODES // _ROW_BLK,),
        in_specs=[_h_spec, _h_spec, _h_spec, _h_spec,
                  _w_spec, _b_spec, _w_spec, _b_spec, _w_spec, _b_spec],
        out_specs=_d_spec,
        out_shape=jax.ShapeDtypeStruct((N_NODES, D), jnp.float32),
    )(x0, x1, a0, a1, wa, ba, wb, bb, wl, bl)


def kernel(x, edge_index, W1a, b1a, W1b, b1b, W2a, b2a, W2b, b2b, Wl, bl):
    src = edge_index[0].astype(jnp.int32)
    dst = edge_index[1].astype(jnp.int32)
    npad = E_PAD - N_EDGES
    src_p = jnp.concatenate([src, jnp.zeros((npad,), jnp.int32)])
    dst_p = jnp.concatenate(
        [dst, N_NODES + (jnp.arange(npad, dtype=jnp.int32) & 15)])
    src_p = src_p.reshape(NS, NCHT, CHUNK)
    dst_p = dst_p.reshape(NS, NCHT, CHUNK)

    x0, x1 = x[:, :H], x[:, H:]

    ba1, bb1 = b1a.reshape(1, D), b1b.reshape(1, D)
    ba2, bb2 = b2a.reshape(1, D), b2b.reshape(1, D)
    blr = bl.reshape(1, D)

    a0, a1 = _sc_aggregate(x0, x1, src_p, dst_p)
    h0, h1 = _mlp2(x0, x1, a0, a1, W1a, ba1, W1b, bb1)
    a20, a21 = _sc_aggregate(h0, h1, src_p, dst_p)
    out = _mlp3(h0, h1, a20, a21, W2a, ba2, W2b, bb2, Wl, blr)
    return out


# speedup vs baseline: 1.0941x; 1.0941x over previous
import functools

import jax
import jax.numpy as jnp
from jax import lax
from jax.experimental import pallas as pl
from jax.experimental.pallas import tpu as pltpu
from jax.experimental.pallas import tpu_sc as plsc

N_NODES = 10000
N_EDGES = 160000
D = 256
H = 128

NS = 16
CHUNK = 88
NCHT = -(-(N_EDGES // NS) // CHUNK)
NBUF = 4
IBUF = 5
EPT = NCHT * CHUNK
E_PAD = EPT * NS
ROWS_PT = 624
ROWS_LAST = N_NODES - (NS - 1) * ROWS_PT
ACC_ROWS = N_NODES + 16

_sc_mesh = plsc.VectorSubcoreMesh(core_axis_name="c", subcore_axis_name="s")


@functools.partial(
    pl.kernel,
    out_type=[jax.ShapeDtypeStruct((N_NODES, H), jnp.float32),
              jax.ShapeDtypeStruct((N_NODES, H), jnp.float32)],
    mesh=_sc_mesh,
    scratch_types=[
        pltpu.VMEM((IBUF, CHUNK), jnp.int32),
        pltpu.VMEM((IBUF, CHUNK), jnp.int32),
        pltpu.VMEM((NBUF, CHUNK, H), jnp.float32),
        pltpu.VMEM_SHARED((ACC_ROWS, H), jnp.float32),
        pltpu.SemaphoreType.DMA,
        pltpu.SemaphoreType.DMA,
        pltpu.SemaphoreType.DMA,
    ],
)
def _sc_aggregate(t0_hbm, t1_hbm, src_hbm, dst_hbm, out0_hbm, out1_hbm,
                  srcb, dstb, rows_v, acc, gsem, ssem, isem):
    c = lax.axis_index("c")
    s = lax.axis_index("s")
    r0 = s * ROWS_PT

    def idx_desc(j, b):
        return (pltpu.make_async_copy(src_hbm.at[s, j], srcb.at[b], isem),
                pltpu.make_async_copy(dst_hbm.at[s, j], dstb.at[b], isem))

    def start_gather(j_b, b):
        @pl.when(c == 0)
        def _():
            pltpu.make_async_copy(t0_hbm.at[srcb.at[j_b]],
                                  rows_v.at[b], gsem).start()

        @pl.when(c == 1)
        def _():
            pltpu.make_async_copy(t1_hbm.at[srcb.at[j_b]],
                                  rows_v.at[b], gsem).start()

    def wait_gather(j_b, b):
        pltpu.make_async_copy(t0_hbm.at[srcb.at[j_b]],
                              rows_v.at[b], gsem).wait()

    def scatter_desc(j_b, b):
        return pltpu.make_async_copy(rows_v.at[b], acc.at[dstb.at[j_b]],
                                     ssem)

    def wait_idx(j, b):
        for d in idx_desc(j, b):
            d.wait()

    def zrow(r, carry):
        for i in range(H // 16):
            rows_v[0, r, pl.ds(i * 16, 16)] = jnp.zeros((16,), jnp.float32)
        return carry

    lax.fori_loop(0, CHUNK, zrow, 0)

    def emit_zero(nrows):
        descs = []
        for k in range(nrows // CHUNK):
            descs.append(pltpu.make_async_copy(
                rows_v.at[0], acc.at[pl.ds(r0 + k * CHUNK, CHUNK)], ssem))
        tail = nrows % CHUNK
        if tail:
            descs.append(pltpu.make_async_copy(
                rows_v.at[0, pl.ds(0, tail)],
                acc.at[pl.ds(r0 + (nrows // CHUNK) * CHUNK, tail)], ssem))
        for d in descs:
            d.start()
        for d in descs:
            d.wait()

    @pl.when(s < NS - 1)
    def _():
        emit_zero(ROWS_PT)

    @pl.when(s == NS - 1)
    def _():
        emit_zero(ROWS_LAST)

    for k in range(IBUF - 1):
        for d in idx_desc(k, k):
            d.start()
    wait_idx(0, 0)
    wait_idx(1, 1)
    wait_idx(2, 2)
    plsc.subcore_barrier()
    start_gather(0, 0)
    start_gather(1, 1)
    start_gather(2, 2)

    def body(j, carry):
        b = lax.rem(j, NBUF)
        jb = lax.rem(j, IBUF)
        wait_gather(jb, b)
        pltpu.async_copy(rows_v.at[b], acc.at[dstb.at[jb]], ssem, add=True)

        @pl.when(j >= 1)
        def _():
            scatter_desc(lax.rem(j + IBUF - 1, IBUF),
                         lax.rem(j + NBUF - 1, NBUF)).wait()

        @pl.when(j + IBUF - 1 <= NCHT - 1)
        def _():
            for d in idx_desc(j + IBUF - 1, lax.rem(j + IBUF - 1, IBUF)):
                d.start()

        @pl.when(j + 3 <= NCHT - 1)
        def _():
            nb = lax.rem(j + 3, IBUF)
            wait_idx(j + 3, nb)
            start_gather(nb, lax.rem(j + 3, NBUF))
        return carry

    lax.fori_loop(0, NCHT, body, 0)
    scatter_desc(lax.rem(NCHT - 1, IBUF), lax.rem(NCHT - 1, NBUF)).wait()
    plsc.subcore_barrier()

    def emit_out(out_hbm, nrows):
        pltpu.sync_copy(acc.at[pl.ds(r0, nrows)],
                        out_hbm.at[pl.ds(r0, nrows)])

    @pl.when((c == 0) & (s < NS - 1))
    def _():
        emit_out(out0_hbm, ROWS_PT)

    @pl.when((c == 0) & (s == NS - 1))
    def _():
        emit_out(out0_hbm, ROWS_LAST)

    @pl.when((c == 1) & (s < NS - 1))
    def _():
        emit_out(out1_hbm, ROWS_PT)

    @pl.when((c == 1) & (s == NS - 1))
    def _():
        emit_out(out1_hbm, ROWS_LAST)


def _mlp2_body(x0_ref, x1_ref, a0_ref, a1_ref, wa_ref, ba_ref, wb_ref, bb_ref,
               o0_ref, o1_ref):
    g0 = x0_ref[...] + a0_ref[...]
    g1 = x1_ref[...] + a1_ref[...]
    wa = wa_ref[...]
    t = jnp.dot(g0, wa[:H], preferred_element_type=jnp.float32)
    t += jnp.dot(g1, wa[H:], preferred_element_type=jnp.float32)
    t = jnp.maximum(t + ba_ref[...], 0.0)
    u = jnp.dot(t, wb_ref[...], preferred_element_type=jnp.float32)
    u = jnp.maximum(u + bb_ref[...], 0.0)
    o0_ref[...] = u[:, :H]
    o1_ref[...] = u[:, H:]


def _mlp3_body(x0_ref, x1_ref, a0_ref, a1_ref, wa_ref, ba_ref, wb_ref, bb_ref,
               wl_ref, bl_ref, o_ref):
    g0 = x0_ref[...] + a0_ref[...]
    g1 = x1_ref[...] + a1_ref[...]
    wa = wa_ref[...]
    t = jnp.dot(g0, wa[:H], preferred_element_type=jnp.float32)
    t += jnp.dot(g1, wa[H:], preferred_element_type=jnp.float32)
    t = jnp.maximum(t + ba_ref[...], 0.0)
    u = jnp.dot(t, wb_ref[...], preferred_element_type=jnp.float32)
    u = jnp.maximum(u + bb_ref[...], 0.0)
    o_ref[...] = jnp.dot(u, wl_ref[...], preferred_element_type=jnp.float32) + bl_ref[...]


_ROW_BLK = 1000
_h_spec = pl.BlockSpec((_ROW_BLK, H), lambda i: (i, 0))
_d_spec = pl.BlockSpec((_ROW_BLK, D), lambda i: (i, 0))
_w_spec = pl.BlockSpec((D, D), lambda i: (0, 0))
_b_spec = pl.BlockSpec((1, D), lambda i: (0, 0))


def _mlp2(x0, x1, a0, a1, wa, ba, wb, bb):
    return pl.pallas_call(
        _mlp2_body,
        grid=(N_NODES // _ROW_BLK,),
        in_specs=[_h_spec, _h_spec, _h_spec, _h_spec,
                  _w_spec, _b_spec, _w_spec, _b_spec],
        out_specs=[_h_spec, _h_spec],
        out_shape=[jax.ShapeDtypeStruct((N_NODES, H), jnp.float32),
                   jax.ShapeDtypeStruct((N_NODES, H), jnp.float32)],
    )(x0, x1, a0, a1, wa, ba, wb, bb)


def _mlp3(x0, x1, a0, a1, wa, ba, wb, bb, wl, bl):
    return pl.pallas_call(
        _mlp3_body,
        grid=(N_NODES // _ROW_BLK,),
        in_specs=[_h_spec, _h_spec, _h_spec, _h_spec,
                  _w_spec, _b_spec, _w_spec, _b_spec, _w_spec, _b_spec],
        out_specs=_d_spec,
        out_shape=jax.ShapeDtypeStruct((N_NODES, D), jnp.float32),
    )(x0, x1, a0, a1, wa, ba, wb, bb, wl, bl)


def kernel(x, edge_index, W1a, b1a, W1b, b1b, W2a, b2a, W2b, b2b, Wl, bl):
    src = edge_index[0].astype(jnp.int32)
    dst = edge_index[1].astype(jnp.int32)
    npad = E_PAD - N_EDGES
    src_p = jnp.concatenate([src, jnp.zeros((npad,), jnp.int32)])
    dst_p = jnp.concatenate(
        [dst, N_NODES + (jnp.arange(npad, dtype=jnp.int32) & 15)])
    src_p = src_p.reshape(NS, NCHT, CHUNK)
    dst_p = dst_p.reshape(NS, NCHT, CHUNK)

    x0, x1 = x[:, :H], x[:, H:]

    ba1, bb1 = b1a.reshape(1, D), b1b.reshape(1, D)
    ba2, bb2 = b2a.reshape(1, D), b2b.reshape(1, D)
    blr = bl.reshape(1, D)

    a0, a1 = _sc_aggregate(x0, x1, src_p, dst_p)
    h0, h1 = _mlp2(x0, x1, a0, a1, W1a, ba1, W1b, bb1)
    a20, a21 = _sc_aggregate(h0, h1, src_p, dst_p)
    out = _mlp3(h0, h1, a20, a21, W2a, ba2, W2b, bb2, Wl, blr)
    return out


# speedup vs baseline: 1.0952x; 1.0010x over previous
import functools

import jax
import jax.numpy as jnp
from jax import lax
from jax.experimental import pallas as pl
from jax.experimental.pallas import tpu as pltpu
from jax.experimental.pallas import tpu_sc as plsc

N_NODES = 10000
N_EDGES = 160000
D = 256
H = 128

NS = 16
CHUNK = 88
NCHT = -(-(N_EDGES // NS) // CHUNK)
NBUF = 4
IBUF = 5
EPT = NCHT * CHUNK
E_PAD = EPT * NS
ROWS_PT = 624
ROWS_LAST = N_NODES - (NS - 1) * ROWS_PT
ACC_ROWS = N_NODES + 16

_sc_mesh = plsc.VectorSubcoreMesh(core_axis_name="c", subcore_axis_name="s")


@functools.partial(
    pl.kernel,
    out_type=[jax.ShapeDtypeStruct((N_NODES, H), jnp.float32),
              jax.ShapeDtypeStruct((N_NODES, H), jnp.float32)],
    mesh=_sc_mesh,
    scratch_types=[
        pltpu.VMEM((IBUF, CHUNK), jnp.int32),
        pltpu.VMEM((IBUF, CHUNK), jnp.int32),
        pltpu.VMEM((NBUF, CHUNK, H), jnp.float32),
        pltpu.VMEM_SHARED((ACC_ROWS, H), jnp.float32),
        pltpu.SemaphoreType.DMA,
        pltpu.SemaphoreType.DMA,
        pltpu.SemaphoreType.DMA,
    ],
)
def _sc_aggregate(t0_hbm, t1_hbm, src_hbm, dst_hbm, out0_hbm, out1_hbm,
                  srcb, dstb, rows_v, acc, gsem, ssem, isem):
    c = lax.axis_index("c")
    s = lax.axis_index("s")
    r0 = s * ROWS_PT

    def idx_desc(j, b):
        return (pltpu.make_async_copy(src_hbm.at[s, j], srcb.at[b], isem),
                pltpu.make_async_copy(dst_hbm.at[s, j], dstb.at[b], isem))

    def start_gather(j_b, b):
        @pl.when(c == 0)
        def _():
            pltpu.make_async_copy(t0_hbm.at[srcb.at[j_b]],
                                  rows_v.at[b], gsem).start()

        @pl.when(c == 1)
        def _():
            pltpu.make_async_copy(t1_hbm.at[srcb.at[j_b]],
                                  rows_v.at[b], gsem).start()

    def wait_gather(j_b, b):
        pltpu.make_async_copy(t0_hbm.at[srcb.at[j_b]],
                              rows_v.at[b], gsem).wait()

    def scatter_desc(j_b, b):
        return pltpu.make_async_copy(rows_v.at[b], acc.at[dstb.at[j_b]],
                                     ssem)

    def wait_idx(j, b):
        for d in idx_desc(j, b):
            d.wait()

    def zrow(r, carry):
        for i in range(H // 16):
            rows_v[0, r, pl.ds(i * 16, 16)] = jnp.zeros((16,), jnp.float32)
        return carry

    lax.fori_loop(0, CHUNK, zrow, 0)

    def emit_zero(nrows):
        descs = []
        for k in range(nrows // CHUNK):
            descs.append(pltpu.make_async_copy(
                rows_v.at[0], acc.at[pl.ds(r0 + k * CHUNK, CHUNK)], ssem))
        tail = nrows % CHUNK
        if tail:
            descs.append(pltpu.make_async_copy(
                rows_v.at[0, pl.ds(0, tail)],
                acc.at[pl.ds(r0 + (nrows // CHUNK) * CHUNK, tail)], ssem))
        for d in descs:
            d.start()
        for d in descs:
            d.wait()

    @pl.when(s < NS - 1)
    def _():
        emit_zero(ROWS_PT)

    @pl.when(s == NS - 1)
    def _():
        emit_zero(ROWS_LAST)

    for k in range(IBUF - 1):
        for d in idx_desc(k, k):
            d.start()
    wait_idx(0, 0)
    wait_idx(1, 1)
    wait_idx(2, 2)
    plsc.subcore_barrier()
    start_gather(0, 0)
    start_gather(1, 1)
    start_gather(2, 2)

    def body(j, carry):
        b = lax.rem(j, NBUF)
        jb = lax.rem(j, IBUF)
        wait_gather(jb, b)
        pltpu.async_copy(rows_v.at[b], acc.at[dstb.at[jb]], ssem, add=True)

        @pl.when(j >= 1)
        def _():
            scatter_desc(lax.rem(j + IBUF - 1, IBUF),
                         lax.rem(j + NBUF - 1, NBUF)).wait()

        @pl.when(j + IBUF - 1 <= NCHT - 1)
        def _():
            for d in idx_desc(j + IBUF - 1, lax.rem(j + IBUF - 1, IBUF)):
                d.start()

        @pl.when(j + 3 <= NCHT - 1)
        def _():
            nb = lax.rem(j + 3, IBUF)
            wait_idx(j + 3, nb)
            start_gather(nb, lax.rem(j + 3, NBUF))
        return carry

    lax.fori_loop(0, NCHT, body, 0)
    scatter_desc(lax.rem(NCHT - 1, IBUF), lax.rem(NCHT - 1, NBUF)).wait()
    plsc.subcore_barrier()

    def emit_out(out_hbm, nrows):
        pltpu.sync_copy(acc.at[pl.ds(r0, nrows)],
                        out_hbm.at[pl.ds(r0, nrows)])

    @pl.when((c == 0) & (s < NS - 1))
    def _():
        emit_out(out0_hbm, ROWS_PT)

    @pl.when((c == 0) & (s == NS - 1))
    def _():
        emit_out(out0_hbm, ROWS_LAST)

    @pl.when((c == 1) & (s < NS - 1))
    def _():
        emit_out(out1_hbm, ROWS_PT)

    @pl.when((c == 1) & (s == NS - 1))
    def _():
        emit_out(out1_hbm, ROWS_LAST)


def _bf(v):
    return v.astype(jnp.bfloat16)


def _mlp2_body(x0_ref, x1_ref, a0_ref, a1_ref, wa_ref, ba_ref, wb_ref, bb_ref,
               o0_ref, o1_ref):
    g0 = x0_ref[...] + a0_ref[...]
    g1 = x1_ref[...] + a1_ref[...]
    wa = wa_ref[...]
    t = jnp.dot(_bf(g0), _bf(wa[:H]), preferred_element_type=jnp.float32)
    t += jnp.dot(_bf(g1), _bf(wa[H:]), preferred_element_type=jnp.float32)
    t = jnp.maximum(t + ba_ref[...], 0.0)
    u = jnp.dot(_bf(t), _bf(wb_ref[...]), preferred_element_type=jnp.float32)
    u = jnp.maximum(u + bb_ref[...], 0.0)
    o0_ref[...] = u[:, :H]
    o1_ref[...] = u[:, H:]


def _mlp3_body(x0_ref, x1_ref, a0_ref, a1_ref, wa_ref, ba_ref, wb_ref, bb_ref,
               wl_ref, bl_ref, o_ref):
    g0 = x0_ref[...] + a0_ref[...]
    g1 = x1_ref[...] + a1_ref[...]
    wa = wa_ref[...]
    t = jnp.dot(_bf(g0), _bf(wa[:H]), preferred_element_type=jnp.float32)
    t += jnp.dot(_bf(g1), _bf(wa[H:]), preferred_element_type=jnp.float32)
    t = jnp.maximum(t + ba_ref[...], 0.0)
    u = jnp.dot(_bf(t), _bf(wb_ref[...]), preferred_element_type=jnp.float32)
    u = jnp.maximum(u + bb_ref[...], 0.0)
    o_ref[...] = jnp.dot(_bf(u), _bf(wl_ref[...]), preferred_element_type=jnp.float32) + bl_ref[...]


_ROW_BLK = 1000
_h_spec = pl.BlockSpec((_ROW_BLK, H), lambda i: (i, 0))
_d_spec = pl.BlockSpec((_ROW_BLK, D), lambda i: (i, 0))
_w_spec = pl.BlockSpec((D, D), lambda i: (0, 0))
_b_spec = pl.BlockSpec((1, D), lambda i: (0, 0))


def _mlp2(x0, x1, a0, a1, wa, ba, wb, bb):
    return pl.pallas_call(
        _mlp2_body,
        grid=(N_NODES // _ROW_BLK,),
        in_specs=[_h_spec, _h_spec, _h_spec, _h_spec,
                  _w_spec, _b_spec, _w_spec, _b_spec],
        out_specs=[_h_spec, _h_spec],
        out_shape=[jax.ShapeDtypeStruct((N_NODES, H), jnp.float32),
                   jax.ShapeDtypeStruct((N_NODES, H), jnp.float32)],
    )(x0, x1, a0, a1, wa, ba, wb, bb)


def _mlp3(x0, x1, a0, a1, wa, ba, wb, bb, wl, bl):
    return pl.pallas_call(
        _mlp3_body,
        grid=(N_NODES // _ROW_BLK,),
        in_specs=[_h_spec, _h_spec, _h_spec, _h_spec,
                  _w_spec, _b_spec, _w_spec, _b_spec, _w_spec, _b_spec],
        out_specs=_d_spec,
        out_shape=jax.ShapeDtypeStruct((N_NODES, D), jnp.float32),
    )(x0, x1, a0, a1, wa, ba, wb, bb, wl, bl)


def kernel(x, edge_index, W1a, b1a, W1b, b1b, W2a, b2a, W2b, b2b, Wl, bl):
    src = edge_index[0].astype(jnp.int32)
    dst = edge_index[1].astype(jnp.int32)
    npad = E_PAD - N_EDGES
    src_p = jnp.concatenate([src, jnp.zeros((npad,), jnp.int32)])
    dst_p = jnp.concatenate(
        [dst, N_NODES + (jnp.arange(npad, dtype=jnp.int32) & 15)])
    src_p = src_p.reshape(NS, NCHT, CHUNK)
    dst_p = dst_p.reshape(NS, NCHT, CHUNK)

    x0, x1 = x[:, :H], x[:, H:]

    ba1, bb1 = b1a.reshape(1, D), b1b.reshape(1, D)
    ba2, bb2 = b2a.reshape(1, D), b2b.reshape(1, D)
    blr = bl.reshape(1, D)

    a0, a1 = _sc_aggregate(x0, x1, src_p, dst_p)
    h0, h1 = _mlp2(x0, x1, a0, a1, W1a, ba1, W1b, bb1)
    a20, a21 = _sc_aggregate(h0, h1, src_p, dst_p)
    out = _mlp3(h0, h1, a20, a21, W2a, ba2, W2b, bb2, Wl, blr)
    return out


# speedup vs baseline: 1.1152x; 1.0182x over previous
import functools

import jax
import jax.numpy as jnp
from jax import lax
from jax.experimental import pallas as pl
from jax.experimental.pallas import tpu as pltpu
from jax.experimental.pallas import tpu_sc as plsc

N_NODES = 10000
N_EDGES = 160000
D = 256
H = 128

NS = 16
CHUNK = 88
NCHT = -(-(N_EDGES // NS) // CHUNK)
NBUF = 4
IBUF = 5
EPT = NCHT * CHUNK
E_PAD = EPT * NS
ROWS_PT = 624
ROWS_LAST = N_NODES - (NS - 1) * ROWS_PT
ACC_ROWS = N_NODES + 16

_sc_mesh = plsc.VectorSubcoreMesh(core_axis_name="c", subcore_axis_name="s")


@functools.partial(
    pl.kernel,
    out_type=[jax.ShapeDtypeStruct((N_NODES, H), jnp.float32),
              jax.ShapeDtypeStruct((N_NODES, H), jnp.float32)],
    mesh=_sc_mesh,
    scratch_types=[
        pltpu.VMEM((IBUF, CHUNK), jnp.int32),
        pltpu.VMEM((IBUF, CHUNK), jnp.int32),
        pltpu.VMEM((NBUF, CHUNK, H), jnp.float32),
        pltpu.VMEM_SHARED((ACC_ROWS, H), jnp.float32),
        pltpu.SemaphoreType.DMA,
        pltpu.SemaphoreType.DMA,
        pltpu.SemaphoreType.DMA,
    ],
)
def _sc_aggregate(t0_hbm, t1_hbm, src_hbm, dst_hbm, out0_hbm, out1_hbm,
                  srcb, dstb, rows_v, acc, gsem, ssem, isem):
    c = lax.axis_index("c")
    s = lax.axis_index("s")
    r0 = s * ROWS_PT

    def idx_desc(j, b):
        return (pltpu.make_async_copy(src_hbm.at[s, j], srcb.at[b], isem),
                pltpu.make_async_copy(dst_hbm.at[s, j], dstb.at[b], isem))

    def start_gather(j_b, b):
        @pl.when(c == 0)
        def _():
            pltpu.make_async_copy(t0_hbm.at[srcb.at[j_b]],
                                  rows_v.at[b], gsem).start()

        @pl.when(c == 1)
        def _():
            pltpu.make_async_copy(t1_hbm.at[srcb.at[j_b]],
                                  rows_v.at[b], gsem).start()

    def wait_gather(j_b, b):
        pltpu.make_async_copy(t0_hbm.at[srcb.at[j_b]],
                              rows_v.at[b], gsem).wait()

    def scatter_desc(j_b, b):
        return pltpu.make_async_copy(rows_v.at[b], acc.at[dstb.at[j_b]],
                                     ssem)

    def wait_idx(j, b):
        for d in idx_desc(j, b):
            d.wait()

    def zrow(r, carry):
        for i in range(H // 16):
            rows_v[0, r, pl.ds(i * 16, 16)] = jnp.zeros((16,), jnp.float32)
        return carry

    lax.fori_loop(0, CHUNK, zrow, 0)

    def emit_zero(nrows):
        descs = []
        for k in range(nrows // CHUNK):
            descs.append(pltpu.make_async_copy(
                rows_v.at[0], acc.at[pl.ds(r0 + k * CHUNK, CHUNK)], ssem))
        tail = nrows % CHUNK
        if tail:
            descs.append(pltpu.make_async_copy(
                rows_v.at[0, pl.ds(0, tail)],
                acc.at[pl.ds(r0 + (nrows // CHUNK) * CHUNK, tail)], ssem))
        for d in descs:
            d.start()
        for d in descs:
            d.wait()

    @pl.when(s < NS - 1)
    def _():
        emit_zero(ROWS_PT)

    @pl.when(s == NS - 1)
    def _():
        emit_zero(ROWS_LAST)

    for k in range(IBUF - 1):
        for d in idx_desc(k, k):
            d.start()
    wait_idx(0, 0)
    wait_idx(1, 1)
    wait_idx(2, 2)
    plsc.subcore_barrier()
    start_gather(0, 0)
    start_gather(1, 1)
    start_gather(2, 2)

    def body(j, carry):
        b = lax.rem(j, NBUF)
        jb = lax.rem(j, IBUF)
        wait_gather(jb, b)
        pltpu.async_copy(rows_v.at[b], acc.at[dstb.at[jb]], ssem, add=True)

        @pl.when(j >= 1)
        def _():
            scatter_desc(lax.rem(j + IBUF - 1, IBUF),
                         lax.rem(j + NBUF - 1, NBUF)).wait()

        @pl.when(j + IBUF - 1 <= NCHT - 1)
        def _():
            for d in idx_desc(j + IBUF - 1, lax.rem(j + IBUF - 1, IBUF)):
                d.start()

        @pl.when(j + 3 <= NCHT - 1)
        def _():
            nb = lax.rem(j + 3, IBUF)
            wait_idx(j + 3, nb)
            start_gather(nb, lax.rem(j + 3, NBUF))
        return carry

    lax.fori_loop(0, NCHT, body, 0)
    scatter_desc(lax.rem(NCHT - 1, IBUF), lax.rem(NCHT - 1, NBUF)).wait()
    plsc.subcore_barrier()

    def emit_out(out_hbm, nrows):
        pltpu.sync_copy(acc.at[pl.ds(r0, nrows)],
                        out_hbm.at[pl.ds(r0, nrows)])

    @pl.when((c == 0) & (s < NS - 1))
    def _():
        emit_out(out0_hbm, ROWS_PT)

    @pl.when((c == 0) & (s == NS - 1))
    def _():
        emit_out(out0_hbm, ROWS_LAST)

    @pl.when((c == 1) & (s < NS - 1))
    def _():
        emit_out(out1_hbm, ROWS_PT)

    @pl.when((c == 1) & (s == NS - 1))
    def _():
        emit_out(out1_hbm, ROWS_LAST)


def _mlp2_body(x0_ref, x1_ref, a0_ref, a1_ref, wa_ref, ba_ref, wb_ref, bb_ref,
               o0_ref, o1_ref):
    g0 = x0_ref[...] + a0_ref[...]
    g1 = x1_ref[...] + a1_ref[...]
    wa = wa_ref[...]
    t = jnp.dot(g0, wa[:H], preferred_element_type=jnp.float32)
    t += jnp.dot(g1, wa[H:], preferred_element_type=jnp.float32)
    t = jnp.maximum(t + ba_ref[...], 0.0)
    u = jnp.dot(t, wb_ref[...], preferred_element_type=jnp.float32)
    u = jnp.maximum(u + bb_ref[...], 0.0)
    o0_ref[...] = u[:, :H]
    o1_ref[...] = u[:, H:]


def _mlp3_body(x0_ref, x1_ref, a0_ref, a1_ref, wa_ref, ba_ref, wb_ref, bb_ref,
               wl_ref, bl_ref, o_ref):
    g0 = x0_ref[...] + a0_ref[...]
    g1 = x1_ref[...] + a1_ref[...]
    wa = wa_ref[...]
    t = jnp.dot(g0, wa[:H], preferred_element_type=jnp.float32)
    t += jnp.dot(g1, wa[H:], preferred_element_type=jnp.float32)
    t = jnp.maximum(t + ba_ref[...], 0.0)
    u = jnp.dot(t, wb_ref[...], preferred_element_type=jnp.float32)
    u = jnp.maximum(u + bb_ref[...], 0.0)
    o_ref[...] = jnp.dot(u, wl_ref[...], preferred_element_type=jnp.float32) + bl_ref[...]


_ROW_BLK = 2000
_h_spec = pl.BlockSpec((_ROW_BLK, H), lambda i: (i, 0))
_d_spec = pl.BlockSpec((_ROW_BLK, D), lambda i: (i, 0))
_w_spec = pl.BlockSpec((D, D), lambda i: (0, 0))
_b_spec = pl.BlockSpec((1, D), lambda i: (0, 0))


def _mlp2(x0, x1, a0, a1, wa, ba, wb, bb):
    return pl.pallas_call(
        _mlp2_body,
        grid=(N_NODES // _ROW_BLK,),
        in_specs=[_h_spec, _h_spec, _h_spec, _h_spec,
                  _w_spec, _b_spec, _w_spec, _b_spec],
        out_specs=[_h_spec, _h_spec],
        out_shape=[jax.ShapeDtypeStruct((N_NODES, H), jnp.float32),
                   jax.ShapeDtypeStruct((N_NODES, H), jnp.float32)],
    )(x0, x1, a0, a1, wa, ba, wb, bb)


def _mlp3(x0, x1, a0, a1, wa, ba, wb, bb, wl, bl):
    return pl.pallas_call(
        _mlp3_body,
        grid=(N_NODES // _ROW_BLK,),
        in_specs=[_h_spec, _h_spec, _h_spec, _h_spec,
                  _w_spec, _b_spec, _w_spec, _b_spec, _w_spec, _b_spec],
        out_specs=_d_spec,
        out_shape=jax.ShapeDtypeStruct((N_NODES, D), jnp.float32),
    )(x0, x1, a0, a1, wa, ba, wb, bb, wl, bl)


def kernel(x, edge_index, W1a, b1a, W1b, b1b, W2a, b2a, W2b, b2b, Wl, bl):
    src = edge_index[0].astype(jnp.int32)
    dst = edge_index[1].astype(jnp.int32)
    npad = E_PAD - N_EDGES
    src_p = jnp.concatenate([src, jnp.zeros((npad,), jnp.int32)])
    dst_p = jnp.concatenate(
        [dst, N_NODES + (jnp.arange(npad, dtype=jnp.int32) & 15)])
    src_p = src_p.reshape(NS, NCHT, CHUNK)
    dst_p = dst_p.reshape(NS, NCHT, CHUNK)

    x0, x1 = x[:, :H], x[:, H:]

    ba1, bb1 = b1a.reshape(1, D), b1b.reshape(1, D)
    ba2, bb2 = b2a.reshape(1, D), b2b.reshape(1, D)
    blr = bl.reshape(1, D)

    a0, a1 = _sc_aggregate(x0, x1, src_p, dst_p)
    h0, h1 = _mlp2(x0, x1, a0, a1, W1a, ba1, W1b, bb1)
    a20, a21 = _sc_aggregate(h0, h1, src_p, dst_p)
    out = _mlp3(h0, h1, a20, a21, W2a, ba2, W2b, bb2, Wl, blr)
    return out
